# baseline (speedup 1.0000x reference)
import functools

import jax
import jax.numpy as jnp
from jax import lax
from jax.experimental import pallas as pl
from jax.experimental.pallas import tpu as pltpu
from jax.experimental.pallas import tpu_sc as plsc

NC = 2
NS = 16
L = 16
NW = NC * NS

B = 16384
D = 64
W = 2 * D
BPW = B // NW
NCHUNK = 4
CROWS = BPW // NCHUNK
CGROUPS = CROWS // L

_mesh = plsc.VectorSubcoreMesh(core_axis_name="c", subcore_axis_name="s")


@functools.partial(
    pl.kernel,
    out_type=jax.ShapeDtypeStruct((B,), jnp.float32),
    mesh=_mesh,
    compiler_params=pltpu.CompilerParams(
        needs_layout_passes=False, use_tc_tiling_on_sc=True),
    scratch_types=[
        pltpu.VMEM((NCHUNK, CROWS), jnp.int32),
        pltpu.VMEM((NCHUNK, CROWS), jnp.int32),
        pltpu.VMEM((NCHUNK, CROWS), jnp.int32),
        pltpu.VMEM((NCHUNK, CROWS), jnp.int32),
        pltpu.VMEM((2, CROWS, W), jnp.float32),
        pltpu.VMEM((2, CROWS, W), jnp.float32),
        pltpu.VMEM((NCHUNK, CROWS), jnp.float32),
        pltpu.VMEM((NCHUNK, CROWS), jnp.float32),
        pltpu.VMEM((L,), jnp.float32),
        pltpu.VMEM((BPW,), jnp.float32),
        pltpu.SemaphoreType.DMA,
        pltpu.SemaphoreType.DMA,
        pltpu.SemaphoreType.DMA,
        pltpu.SemaphoreType.DMA,
    ],
)
def _rec_kernel(users, movies, U2, M2, bu, bm, b0v, out,
                uidx, midx, urow, mrow, urows, mrows, burow, bmrow, b0_v,
                outbuf, sem_row0, sem_row1, sem_bu, sem_bm):
    wid = lax.axis_index("s") * NC + lax.axis_index("c")
    base = wid * BPW
    for c in range(NCHUNK):
        pltpu.sync_copy(users.at[pl.ds(base + c * CROWS, CROWS)], uidx.at[c])
        pltpu.sync_copy(movies.at[pl.ds(base + c * CROWS, CROWS)], midx.at[c])
    for c in range(NCHUNK):
        for v in range(CROWS // L):
            sl = pl.ds(v * L, L)
            uv = uidx[c, sl]
            mv = midx[c, sl]
            urow[c, sl] = ((uv >> 12) << 11) | (uv & (_TH - 1))
            mrow[c, sl] = ((mv >> 12) << 11) | (mv & (_TH - 1))
    bias_waits = []
    for c in range(NCHUNK):
        wbu = pltpu.async_copy(bu.at[uidx.at[c]], burow.at[c], sem_bu)
        wbm = pltpu.async_copy(bm.at[midx.at[c]], bmrow.at[c], sem_bm)
        bias_waits.append((wbu, wbm))
    pltpu.sync_copy(b0v, b0_v)
    b0x = b0_v[...]

    sems = [sem_row0, sem_row1]
    waits = [None, None]

    def start(chunk, buf):
        cu = pltpu.async_copy(U2.at[urow.at[chunk]], urows.at[buf], sems[buf])
        cm = pltpu.async_copy(M2.at[mrow.at[chunk]], mrows.at[buf], sems[buf])
        return (cu, cm)

    waits[0] = start(0, 0)
    for chunk in range(NCHUNK):
        buf = chunk % 2
        if chunk + 1 < NCHUNK:
            waits[1 - buf] = start(chunk + 1, 1 - buf)
        cu, cm = waits[buf]
        cu.wait()
        cm.wait()
        if chunk == 0:
            for wbu, wbm in bias_waits:
                wbu.wait()
                wbm.wait()
        ub = urows.at[buf]
        mb = mrows.at[buf]
        bc = burow.at[chunk]
        mc = bmrow.at[chunk]
        uix = uidx.at[chunk]
        mix = midx.at[chunk]

        def group(g, carry):
            rows = g * L + lax.iota(jnp.int32, L)
            usel = ((uix[pl.ds(g * L, L)] >> 11) & 1) * D
            msel = ((mix[pl.ds(g * L, L)] >> 11) & 1) * D
            acc = jnp.zeros((L,), jnp.float32)
            for j in range(D):
                uv = plsc.load_gather(ub, [rows, usel + j])
                mv = plsc.load_gather(mb, [rows, msel + j])
                acc = acc + uv * mv
            r = acc + bc[pl.ds(g * L, L)] + mc[pl.ds(g * L, L)] + b0x
            outbuf[pl.ds(chunk * CROWS + g * L, L)] = 5.0 / (1.0 + jnp.exp(-r))
            return carry

        lax.fori_loop(0, CGROUPS, group, 0)

    pltpu.sync_copy(outbuf, out.at[pl.ds(base, BPW)])


_TBLK = 4096
_TH = _TBLK // 2


def _transpose_body(xt_ref, o_ref):
    xb = xt_ref[...]
    eye = jnp.eye(D, dtype=jnp.float32)
    dims = (((0,), (0,)), ((), ()))
    o_ref[:, 0:D] = jax.lax.dot_general(
        xb[:, 0:_TH], eye, dims, preferred_element_type=jnp.float32)
    o_ref[:, D:W] = jax.lax.dot_general(
        xb[:, _TH:_TBLK], eye, dims, preferred_element_type=jnp.float32)


def _pack_rows(xt):
    n = xt.shape[1]
    grid = (n + _TBLK - 1) // _TBLK
    return pl.pallas_call(
        _transpose_body,
        grid=(grid,),
        in_specs=[pl.BlockSpec((D, _TBLK), lambda i: (0, i))],
        out_specs=pl.BlockSpec((_TH, W), lambda i: (i, 0)),
        out_shape=jax.ShapeDtypeStruct((grid * _TH, W), jnp.float32),
    )(xt)


def kernel(users, movies, U, M, bu, bm, b0):
    users = users.astype(jnp.int32)
    movies = movies.astype(jnp.int32)
    U2 = _pack_rows(U.T)
    M2 = _pack_rows(M.T)
    b0v = jnp.full((L,), b0, jnp.float32)
    return _rec_kernel(users, movies, U2, M2, bu, bm, b0v)


# speedup vs baseline: 1.3424x; 1.3424x over previous
import functools

import jax
import jax.numpy as jnp
from jax import lax
from jax.experimental import pallas as pl
from jax.experimental.pallas import tpu as pltpu
from jax.experimental.pallas import tpu_sc as plsc

NC = 2
NS = 16
L = 16
NW = NC * NS

B = 16384
D = 64
W = 2 * D
BPW = B // NW
NCHUNK = 4
CROWS = BPW // NCHUNK
CGROUPS = CROWS // L

_mesh = plsc.VectorSubcoreMesh(core_axis_name="c", subcore_axis_name="s")


@functools.partial(
    pl.kernel,
    out_type=jax.ShapeDtypeStruct((B,), jnp.float32),
    mesh=_mesh,
    compiler_params=pltpu.CompilerParams(
        needs_layout_passes=False, use_tc_tiling_on_sc=True),
    scratch_types=[
        pltpu.VMEM((NCHUNK, CROWS), jnp.int32),
        pltpu.VMEM((NCHUNK, CROWS), jnp.int32),
        pltpu.VMEM((NCHUNK, CROWS), jnp.int32),
        pltpu.VMEM((NCHUNK, CROWS), jnp.int32),
        pltpu.VMEM((2, CROWS, W), jnp.float32),
        pltpu.VMEM((2, CROWS, W), jnp.float32),
        pltpu.VMEM((NCHUNK, CROWS), jnp.float32),
        pltpu.VMEM((NCHUNK, CROWS), jnp.float32),
        pltpu.VMEM((L,), jnp.float32),
        pltpu.VMEM((BPW,), jnp.float32),
        pltpu.SemaphoreType.DMA,
        pltpu.SemaphoreType.DMA,
        pltpu.SemaphoreType.DMA,
        pltpu.SemaphoreType.DMA,
    ],
)
def _rec_kernel(users, movies, U2, M2, bu, bm, b0v, out,
                uidx, midx, urow, mrow, urows, mrows, burow, bmrow, b0_v,
                outbuf, sem_row0, sem_row1, sem_bu, sem_bm):
    wid = lax.axis_index("s") * NC + lax.axis_index("c")
    base = wid * BPW
    for c in range(NCHUNK):
        pltpu.sync_copy(users.at[pl.ds(base + c * CROWS, CROWS)], uidx.at[c])
        pltpu.sync_copy(movies.at[pl.ds(base + c * CROWS, CROWS)], midx.at[c])
    for c in range(NCHUNK):
        for v in range(CROWS // L):
            sl = pl.ds(v * L, L)
            uv = uidx[c, sl]
            mv = midx[c, sl]
            urow[c, sl] = ((uv >> _SHB) << _SHH) | (uv & (_TH - 1))
            mrow[c, sl] = ((mv >> _SHB) << _SHH) | (mv & (_TH - 1))
    bias_waits = []
    for c in range(NCHUNK):
        wbu = pltpu.async_copy(bu.at[uidx.at[c]], burow.at[c], sem_bu)
        wbm = pltpu.async_copy(bm.at[midx.at[c]], bmrow.at[c], sem_bm)
        bias_waits.append((wbu, wbm))
    pltpu.sync_copy(b0v, b0_v)
    b0x = b0_v[...]

    sems = [sem_row0, sem_row1]
    waits = [None, None]

    def start(chunk, buf):
        cu = pltpu.async_copy(U2.at[urow.at[chunk]], urows.at[buf], sems[buf])
        cm = pltpu.async_copy(M2.at[mrow.at[chunk]], mrows.at[buf], sems[buf])
        return (cu, cm)

    waits[0] = start(0, 0)
    for chunk in range(NCHUNK):
        buf = chunk % 2
        if chunk + 1 < NCHUNK:
            waits[1 - buf] = start(chunk + 1, 1 - buf)
        cu, cm = waits[buf]
        cu.wait()
        cm.wait()
        if chunk == 0:
            for wbu, wbm in bias_waits:
                wbu.wait()
                wbm.wait()
        ub = urows.at[buf]
        mb = mrows.at[buf]
        bc = burow.at[chunk]
        mc = bmrow.at[chunk]
        uix = uidx.at[chunk]
        mix = midx.at[chunk]

        def group(g, carry):
            rows = g * L + lax.iota(jnp.int32, L)
            usel = ((uix[pl.ds(g * L, L)] >> _SHH) & 1) * D
            msel = ((mix[pl.ds(g * L, L)] >> _SHH) & 1) * D
            acc = jnp.zeros((L,), jnp.float32)
            for j in range(D):
                uv = plsc.load_gather(ub, [rows, usel + j])
                mv = plsc.load_gather(mb, [rows, msel + j])
                acc = acc + uv * mv
            r = acc + bc[pl.ds(g * L, L)] + mc[pl.ds(g * L, L)] + b0x
            outbuf[pl.ds(chunk * CROWS + g * L, L)] = 5.0 / (1.0 + jnp.exp(-r))
            return carry

        lax.fori_loop(0, CGROUPS, group, 0)

    pltpu.sync_copy(outbuf, out.at[pl.ds(base, BPW)])


_TBLK = 16384
_TH = _TBLK // 2
_SHB = _TBLK.bit_length() - 1
_SHH = _TH.bit_length() - 1


def _transpose_body(xt_ref, o_ref):
    xb = xt_ref[...]
    eye = jnp.eye(D, dtype=jnp.float32)
    dims = (((0,), (0,)), ((), ()))
    o_ref[:, 0:D] = jax.lax.dot_general(
        xb[:, 0:_TH], eye, dims, preferred_element_type=jnp.float32)
    o_ref[:, D:W] = jax.lax.dot_general(
        xb[:, _TH:_TBLK], eye, dims, preferred_element_type=jnp.float32)


def _pack_rows(xt):
    n = xt.shape[1]
    grid = (n + _TBLK - 1) // _TBLK
    return pl.pallas_call(
        _transpose_body,
        grid=(grid,),
        in_specs=[pl.BlockSpec((D, _TBLK), lambda i: (0, i))],
        out_specs=pl.BlockSpec((_TH, W), lambda i: (i, 0)),
        out_shape=jax.ShapeDtypeStruct((grid * _TH, W), jnp.float32),
    )(xt)


def kernel(users, movies, U, M, bu, bm, b0):
    users = users.astype(jnp.int32)
    movies = movies.astype(jnp.int32)
    U2 = _pack_rows(U.T)
    M2 = _pack_rows(M.T)
    b0v = jnp.full((L,), b0, jnp.float32)
    return _rec_kernel(users, movies, U2, M2, bu, bm, b0v)


# speedup vs baseline: 1.3829x; 1.0302x over previous
import functools

import jax
import jax.numpy as jnp
from jax import lax
from jax.experimental import pallas as pl
from jax.experimental.pallas import tpu as pltpu
from jax.experimental.pallas import tpu_sc as plsc

NC = 2
NS = 16
L = 16
NW = NC * NS

B = 16384
D = 64
W = 2 * D
BPW = B // NW
NCHUNK = 4
CROWS = BPW // NCHUNK
CGROUPS = CROWS // L

_mesh = plsc.VectorSubcoreMesh(core_axis_name="c", subcore_axis_name="s")


@functools.partial(
    pl.kernel,
    out_type=jax.ShapeDtypeStruct((B,), jnp.float32),
    mesh=_mesh,
    compiler_params=pltpu.CompilerParams(
        needs_layout_passes=False, use_tc_tiling_on_sc=True),
    scratch_types=[
        pltpu.VMEM((NCHUNK, CROWS), jnp.int32),
        pltpu.VMEM((NCHUNK, CROWS), jnp.int32),
        pltpu.VMEM((NCHUNK, CROWS), jnp.int32),
        pltpu.VMEM((NCHUNK, CROWS), jnp.int32),
        pltpu.VMEM((2, CROWS, W), jnp.float32),
        pltpu.VMEM((2, CROWS, W), jnp.float32),
        pltpu.VMEM((NCHUNK, CROWS), jnp.float32),
        pltpu.VMEM((NCHUNK, CROWS), jnp.float32),
        pltpu.VMEM((L,), jnp.float32),
        pltpu.VMEM((BPW,), jnp.float32),
        pltpu.SemaphoreType.DMA,
        pltpu.SemaphoreType.DMA,
        pltpu.SemaphoreType.DMA,
        pltpu.SemaphoreType.DMA,
    ],
)
def _rec_kernel(users, movies, U2, M2, bu, bm, b0v, out,
                uidx, midx, urow, mrow, urows, mrows, burow, bmrow, b0_v,
                outbuf, sem_row0, sem_row1, sem_bu, sem_bm):
    wid = lax.axis_index("s") * NC + lax.axis_index("c")
    base = wid * BPW
    for c in range(NCHUNK):
        pltpu.sync_copy(users.at[pl.ds(base + c * CROWS, CROWS)], uidx.at[c])
        pltpu.sync_copy(movies.at[pl.ds(base + c * CROWS, CROWS)], midx.at[c])
    for c in range(NCHUNK):
        for v in range(CROWS // L):
            sl = pl.ds(v * L, L)
            uv = uidx[c, sl]
            mv = midx[c, sl]
            urow[c, sl] = ((uv >> _SHB) << _SHH) | (uv & (_TH - 1))
            mrow[c, sl] = ((mv >> _SHB) << _SHH) | (mv & (_TH - 1))
    bias_waits = []
    for c in range(NCHUNK):
        wbu = pltpu.async_copy(bu.at[uidx.at[c]], burow.at[c], sem_bu)
        wbm = pltpu.async_copy(bm.at[midx.at[c]], bmrow.at[c], sem_bm)
        bias_waits.append((wbu, wbm))
    pltpu.sync_copy(b0v, b0_v)
    b0x = b0_v[...]

    sems = [sem_row0, sem_row1]
    waits = [None, None]

    def start(chunk, buf):
        cu = pltpu.async_copy(U2.at[urow.at[chunk]], urows.at[buf], sems[buf])
        cm = pltpu.async_copy(M2.at[mrow.at[chunk]], mrows.at[buf], sems[buf])
        return (cu, cm)

    waits[0] = start(0, 0)
    for chunk in range(NCHUNK):
        buf = chunk % 2
        if chunk + 1 < NCHUNK:
            waits[1 - buf] = start(chunk + 1, 1 - buf)
        cu, cm = waits[buf]
        cu.wait()
        cm.wait()
        if chunk == 0:
            for wbu, wbm in bias_waits:
                wbu.wait()
                wbm.wait()
        ub = urows.at[buf]
        mb = mrows.at[buf]
        bc = burow.at[chunk]
        mc = bmrow.at[chunk]
        uix = uidx.at[chunk]
        mix = midx.at[chunk]

        def group(g, carry):
            rows = g * L + lax.iota(jnp.int32, L)
            usel = ((uix[pl.ds(g * L, L)] >> _SHH) & 1) * D
            msel = ((mix[pl.ds(g * L, L)] >> _SHH) & 1) * D
            acc = jnp.zeros((L,), jnp.float32)
            for j in range(D):
                uv = plsc.load_gather(ub, [rows, usel + j])
                mv = plsc.load_gather(mb, [rows, msel + j])
                acc = acc + uv * mv
            r = acc + bc[pl.ds(g * L, L)] + mc[pl.ds(g * L, L)] + b0x
            outbuf[pl.ds(chunk * CROWS + g * L, L)] = 5.0 / (1.0 + jnp.exp(-r))
            return carry

        lax.fori_loop(0, CGROUPS, group, 0)

    pltpu.sync_copy(outbuf, out.at[pl.ds(base, BPW)])


_TBLK = 32768
_TH = _TBLK // 2
_SHB = _TBLK.bit_length() - 1
_SHH = _TH.bit_length() - 1


def _transpose_body(xt_ref, o_ref):
    xb = xt_ref[...]
    eye = jnp.eye(D, dtype=jnp.float32)
    dims = (((0,), (0,)), ((), ()))
    o_ref[:, 0:D] = jax.lax.dot_general(
        xb[:, 0:_TH], eye, dims, preferred_element_type=jnp.float32)
    o_ref[:, D:W] = jax.lax.dot_general(
        xb[:, _TH:_TBLK], eye, dims, preferred_element_type=jnp.float32)


def _pack_rows(xt):
    n = xt.shape[1]
    grid = (n + _TBLK - 1) // _TBLK
    return pl.pallas_call(
        _transpose_body,
        grid=(grid,),
        in_specs=[pl.BlockSpec((D, _TBLK), lambda i: (0, i))],
        out_specs=pl.BlockSpec((_TH, W), lambda i: (i, 0)),
        out_shape=jax.ShapeDtypeStruct((grid * _TH, W), jnp.float32),
    )(xt)


def kernel(users, movies, U, M, bu, bm, b0):
    users = users.astype(jnp.int32)
    movies = movies.astype(jnp.int32)
    U2 = _pack_rows(U.T)
    M2 = _pack_rows(M.T)
    b0v = jnp.full((L,), b0, jnp.float32)
    return _rec_kernel(users, movies, U2, M2, bu, bm, b0v)
